# baseline (speedup 1.0000x reference)
import jax
from jax.experimental import pallas as pl
from jax.experimental.pallas import tpu as pltpu

_CHUNKS = 14


def _dma_body(x_ref, o_ref, sems):
    copies = [
        pltpu.make_async_copy(x_ref.at[i], o_ref.at[i], sems.at[i])
        for i in range(_CHUNKS)
    ]
    for cp in copies:
        cp.start()
    for cp in copies:
        cp.wait()


def kernel(x, center):
    del center
    xt = x.transpose(2, 3, 0, 1)
    yt = pl.pallas_call(
        _dma_body,
        in_specs=[pl.BlockSpec(memory_space=pltpu.MemorySpace.HBM)],
        out_specs=pl.BlockSpec(memory_space=pltpu.MemorySpace.HBM),
        out_shape=jax.ShapeDtypeStruct((14, 14, 256, 768), x.dtype),
        scratch_shapes=[pltpu.SemaphoreType.DMA((_CHUNKS,))],
    )(xt)
    return yt.transpose(2, 3, 0, 1)


# speedup vs baseline: 45.6733x; 45.6733x over previous
import jax
import jax.numpy as jnp
from jax.experimental import pallas as pl
from jax.experimental.pallas import tpu as pltpu

_ROWS, _COLS = 50176, 768
_BLK = 7168
_N = _ROWS // _BLK
_NBUF = 2


def _dma_body(x_hbm, o_hbm, bufs, in_sems, out_sems):
    def in_cp(k):
        return pltpu.make_async_copy(
            x_hbm.at[pl.ds(k * _BLK, _BLK)], bufs.at[k % _NBUF],
            in_sems.at[k % _NBUF],
        )

    def out_cp(k):
        return pltpu.make_async_copy(
            bufs.at[k % _NBUF], o_hbm.at[pl.ds(k * _BLK, _BLK)],
            out_sems.at[k % _NBUF],
        )

    in_cp(0).start()
    for k in range(_N):
        in_cp(k).wait()
        if k + 1 < _N:
            if k + 1 - _NBUF >= 0:
                out_cp(k + 1 - _NBUF).wait()
            in_cp(k + 1).start()
        out_cp(k).start()
    for k in range(max(0, _N - _NBUF), _N):
        out_cp(k).wait()


def kernel(x, center):
    del center
    flat = x.transpose(2, 3, 0, 1).reshape(_ROWS, _COLS)
    yt = pl.pallas_call(
        _dma_body,
        in_specs=[pl.BlockSpec(memory_space=pltpu.MemorySpace.HBM)],
        out_specs=pl.BlockSpec(memory_space=pltpu.MemorySpace.HBM),
        out_shape=jax.ShapeDtypeStruct((_ROWS, _COLS), x.dtype),
        scratch_shapes=[
            pltpu.VMEM((_NBUF, _BLK, _COLS), jnp.float32),
            pltpu.SemaphoreType.DMA((_NBUF,)),
            pltpu.SemaphoreType.DMA((_NBUF,)),
        ],
    )(flat)
    return yt.reshape(14, 14, 256, 768).transpose(2, 3, 0, 1)
